# Initial kernel scaffold; baseline (speedup 1.0000x reference)
import jax
import jax.numpy as jnp
from jax.experimental import pallas as pl


def kernel(x, weight):
    raise NotImplementedError("write your pallas kernel here")


# speedup vs baseline: 5.0810x; 5.0810x over previous
import functools

import jax
import jax.numpy as jnp
from jax import lax
from jax.experimental import pallas as pl
from jax.experimental.pallas import tpu as pltpu
from jax.experimental.pallas import tpu_sc as plsc

_NW = 32
_C = 2048


@functools.partial(jax.jit, static_argnums=(2, 3))
def _embed(x_flat, wcols, n, d):
    per_w = n // _NW
    n_chunks = per_w // _C
    n_t = _C // 16
    mesh = plsc.VectorSubcoreMesh(core_axis_name="c", subcore_axis_name="s")

    @functools.partial(
        pl.kernel,
        out_type=jax.ShapeDtypeStruct((n * d,), jnp.float32),
        mesh=mesh,
        scratch_types=[
            pltpu.VMEM((_C,), jnp.int32),
            pltpu.VMEM((_C * d,), jnp.float32),
            pltpu.VMEM((d, 16), jnp.float32),
        ],
        compiler_params=pltpu.CompilerParams(
            use_tc_tiling_on_sc=False, needs_layout_passes=False
        ),
    )
    def k(w_hbm, x_hbm, out_hbm, xv, outv, wv):
        wid = lax.axis_index("s") * 2 + lax.axis_index("c")
        base = wid * per_w
        pltpu.sync_copy(w_hbm, wv)
        wc = [wv[c] for c in range(d)]
        iota = lax.iota(jnp.int32, 16)
        sc_base = [iota * d + c for c in range(d)]

        def chunk_body(i, carry):
            pltpu.sync_copy(x_hbm.at[pl.ds(base + i * _C, _C)], xv)

            def t_body(t, carry2):
                xs = xv[pl.ds(pl.multiple_of(t * 16, 16), 16)]
                toff = t * (16 * d)
                for c in range(d):
                    vals = jnp.take_along_axis(wc[c], xs, axis=0)
                    plsc.store_scatter(outv, [sc_base[c] + toff], vals)
                return carry2

            lax.fori_loop(0, n_t, t_body, 0, unroll=2)
            pltpu.sync_copy(
                outv, out_hbm.at[pl.ds((base + i * _C) * d, _C * d)]
            )
            return carry

        lax.fori_loop(0, n_chunks, chunk_body, 0)

    return k(wcols, x_flat)


def kernel(x, weight):
    b, s = x.shape
    v, d = weight.shape
    n = b * s
    x_flat = x.reshape(n).astype(jnp.int32)
    wcols = jnp.zeros((d, 16), jnp.float32).at[:, :v].set(weight.T)
    out = _embed(x_flat, wcols, n, d)
    return out.reshape(b, s, d)


# speedup vs baseline: 6.5007x; 1.2794x over previous
import functools

import jax
import jax.numpy as jnp
from jax import lax
from jax.experimental import pallas as pl
from jax.experimental.pallas import tpu as pltpu
from jax.experimental.pallas import tpu_sc as plsc

_NW = 32
_R = 8


@jax.jit
def _embed(x, wcols):
    b, s = x.shape
    d = wcols.shape[0]
    rows_per_w = b // _NW
    n_chunks = rows_per_w // _R
    n_t = (_R * s) // 16
    mesh = plsc.VectorSubcoreMesh(core_axis_name="c", subcore_axis_name="s")

    @functools.partial(
        pl.kernel,
        out_type=jax.ShapeDtypeStruct((b, s, d), jnp.float32),
        mesh=mesh,
        scratch_types=[
            pltpu.VMEM((_R, s), jnp.int32),
            pltpu.VMEM((_R, s, d), jnp.float32),
            pltpu.VMEM((d, 16), jnp.float32),
        ],
        compiler_params=pltpu.CompilerParams(
            use_tc_tiling_on_sc=False, needs_layout_passes=False
        ),
    )
    def k(w_hbm, x_hbm, out_hbm, xv, outv, wv):
        wid = lax.axis_index("s") * 2 + lax.axis_index("c")
        r0 = wid * rows_per_w
        pltpu.sync_copy(w_hbm, wv)
        wc = [wv[c] for c in range(d)]
        iota = lax.iota(jnp.int32, 16)
        cvec = [jnp.full((16,), c, jnp.int32) for c in range(d)]

        def chunk_body(i, carry):
            r = r0 + i * _R
            pltpu.sync_copy(x_hbm.at[pl.ds(r, _R)], xv)

            def t_body(t, carry2):
                p = t * 16 + iota
                i0 = p // s
                i1 = p - i0 * s
                xs = plsc.load_gather(xv, [i0, i1])
                for c in range(d):
                    vals = jnp.take_along_axis(wc[c], xs, axis=0)
                    plsc.store_scatter(outv, [i0, i1, cvec[c]], vals)
                return carry2

            lax.fori_loop(0, n_t, t_body, 0, unroll=2)
            pltpu.sync_copy(outv, out_hbm.at[pl.ds(r, _R)])
            return carry

        lax.fori_loop(0, n_chunks, chunk_body, 0)

    return k(wcols, x)


def kernel(x, weight):
    v, d = weight.shape
    wcols = jnp.zeros((d, 16), jnp.float32).at[:, :v].set(weight.T)
    return _embed(x.astype(jnp.int32), wcols)


# speedup vs baseline: 84.9769x; 13.0719x over previous
import functools

import jax
import jax.numpy as jnp
from jax import lax
from jax.experimental import pallas as pl
from jax.experimental.pallas import tpu as pltpu
from jax.experimental.pallas import tpu_sc as plsc

_NW = 32
_BW = 512
_SC = 25


@jax.jit
def _embed(xt, wcols):
    s, b = xt.shape
    d = wcols.shape[0]
    n_chunks = s // _SC
    mesh = plsc.VectorSubcoreMesh(core_axis_name="c", subcore_axis_name="s")

    @functools.partial(
        pl.kernel,
        out_type=jax.ShapeDtypeStruct((d, s, b), jnp.float32),
        mesh=mesh,
        scratch_types=[
            pltpu.VMEM((_SC, _BW), jnp.int32),
            pltpu.VMEM((d, _SC, _BW), jnp.float32),
            pltpu.VMEM((d, 16), jnp.float32),
        ],
        compiler_params=pltpu.CompilerParams(
            use_tc_tiling_on_sc=False, needs_layout_passes=False
        ),
    )
    def k(w_hbm, x_hbm, out_hbm, xv, outv, wv):
        wid = lax.axis_index("s") * 2 + lax.axis_index("c")
        b0 = wid * _BW
        pltpu.sync_copy(w_hbm, wv)
        wc = [wv[c] for c in range(d)]

        def chunk_body(i, carry):
            s0 = i * _SC
            pltpu.sync_copy(
                x_hbm.at[pl.ds(s0, _SC), pl.ds(b0, _BW)], xv
            )

            def row_body(sl, carry2):
                def blk_body(t, carry3):
                    off = t * 16
                    xs = xv[sl, pl.ds(off, 16)]
                    for c in range(d):
                        vals = jnp.take_along_axis(wc[c], xs, axis=0)
                        outv[c, sl, pl.ds(off, 16)] = vals
                    return carry3

                lax.fori_loop(0, _BW // 16, blk_body, 0, unroll=2)
                return carry2

            lax.fori_loop(0, _SC, row_body, 0)
            pltpu.sync_copy(
                outv, out_hbm.at[:, pl.ds(s0, _SC), pl.ds(b0, _BW)]
            )
            return carry

        lax.fori_loop(0, n_chunks, chunk_body, 0)

    return k(wcols, xt)


def kernel(x, weight):
    v, d = weight.shape
    wcols = jnp.zeros((d, 16), jnp.float32).at[:, :v].set(weight.T)
    xt = jnp.transpose(x.astype(jnp.int32))
    out_t = _embed(xt, wcols)
    return jnp.transpose(out_t, (2, 1, 0))


# speedup vs baseline: 113.2445x; 1.3327x over previous
import functools

import jax
import jax.numpy as jnp
from jax import lax
from jax.experimental import pallas as pl
from jax.experimental.pallas import tpu as pltpu
from jax.experimental.pallas import tpu_sc as plsc

_NW = 32
_ST = 25
_BT = 128
_BTW = 4
_STC = 5


@jax.jit
def _embed(xq, wcols):
    d = wcols.shape[0]
    mesh = plsc.VectorSubcoreMesh(core_axis_name="c", subcore_axis_name="s")

    @functools.partial(
        pl.kernel,
        out_type=jax.ShapeDtypeStruct((d, _ST, _BT, 8, 128), jnp.float32),
        mesh=mesh,
        scratch_types=[
            pltpu.VMEM((_STC, 8, 128), jnp.int32),
            pltpu.VMEM((d, _STC, 8, 128), jnp.float32),
            pltpu.VMEM((d, 16), jnp.float32),
        ],
        compiler_params=pltpu.CompilerParams(
            use_tc_tiling_on_sc=False, needs_layout_passes=False
        ),
    )
    def k(w_hbm, x_hbm, out_hbm, xv, outv, wv):
        wid = lax.axis_index("s") * 2 + lax.axis_index("c")
        bt0 = wid * _BTW
        pltpu.sync_copy(w_hbm, wv)
        wc = [wv[c] for c in range(d)]

        def bt_body(bt_l, carry):
            bt = bt0 + bt_l

            def stc_body(i, carry2):
                st0 = i * _STC
                pltpu.sync_copy(x_hbm.at[pl.ds(st0, _STC), bt], xv)

                def st_body(sl, carry3):
                    for si in range(8):
                        for bq in range(8):
                            xs = xv[sl, si, pl.ds(bq * 16, 16)]
                            for c in range(d):
                                vals = jnp.take_along_axis(wc[c], xs, axis=0)
                                outv[c, sl, si, pl.ds(bq * 16, 16)] = vals
                    return carry3

                lax.fori_loop(0, _STC, st_body, 0)
                pltpu.sync_copy(
                    outv, out_hbm.at[:, pl.ds(st0, _STC), bt]
                )
                return carry2

            lax.fori_loop(0, _ST // _STC, stc_body, 0)
            return carry

        lax.fori_loop(0, _BTW, bt_body, 0)

    return k(wcols, xq)


def kernel(x, weight):
    v, d = weight.shape
    wcols = jnp.zeros((d, 16), jnp.float32).at[:, :v].set(weight.T)
    xq = (
        x.astype(jnp.int32)
        .reshape(_BT, 128, _ST, 8)
        .transpose(2, 0, 3, 1)
    )
    out_q = _embed(xq, wcols)
    return (
        out_q.transpose(2, 4, 1, 3, 0)
        .reshape(16384, 200, d)
    )


# speedup vs baseline: 139.3634x; 1.2306x over previous
import functools

import jax
import jax.numpy as jnp
from jax import lax
from jax.experimental import pallas as pl
from jax.experimental.pallas import tpu as pltpu
from jax.experimental.pallas import tpu_sc as plsc

_NW = 32
_ST = 25
_BT = 128
_BTW = 4
_STC = 5


@jax.jit
def _embed(xq, wcols):
    d = wcols.shape[0]
    mesh = plsc.VectorSubcoreMesh(core_axis_name="c", subcore_axis_name="s")

    n_stc = _ST // _STC
    n_chunks = _BTW * n_stc

    @functools.partial(
        pl.kernel,
        out_type=jax.ShapeDtypeStruct((d, _ST, _BT, 8, 128), jnp.float32),
        mesh=mesh,
        scratch_types=[
            pltpu.VMEM((2, _STC, 8, 128), jnp.int32),
            pltpu.VMEM((2, d, _STC, 8, 128), jnp.float32),
            pltpu.VMEM((d, 16), jnp.float32),
            pltpu.SemaphoreType.DMA,
            pltpu.SemaphoreType.DMA,
            pltpu.SemaphoreType.DMA,
            pltpu.SemaphoreType.DMA,
        ],
        compiler_params=pltpu.CompilerParams(
            use_tc_tiling_on_sc=False, needs_layout_passes=False
        ),
    )
    def k(w_hbm, x_hbm, out_hbm, xv, outv, wv, si0, si1, so0, so1):
        in_sems = (si0, si1)
        out_sems = (so0, so1)
        wid = lax.axis_index("s") * 2 + lax.axis_index("c")
        bt0 = wid * _BTW
        pltpu.sync_copy(w_hbm, wv)
        wc = [wv[c] for c in range(d)]

        def x_src(ci):
            bt = bt0 + ci // n_stc
            st0 = (ci % n_stc) * _STC
            return x_hbm.at[pl.ds(st0, _STC), bt]

        def out_dst(ci):
            bt = bt0 + ci // n_stc
            st0 = (ci % n_stc) * _STC
            return out_hbm.at[:, pl.ds(st0, _STC), bt]

        for b in range(2):
            pltpu.make_async_copy(x_src(b), xv.at[b], in_sems[b]).start()

        def body(j, carry):
            for b in range(2):
                ci = 2 * j + b

                @pl.when(ci >= 2)
                def _():
                    pltpu.make_async_copy(
                        outv.at[b], out_dst(ci - 2), out_sems[b]
                    ).wait()

                pltpu.make_async_copy(x_src(ci), xv.at[b], in_sems[b]).wait()

                def st_body(sl, carry3):
                    for si in range(8):
                        for bq in range(8):
                            xs = xv[b, sl, si, pl.ds(bq * 16, 16)]
                            for c in range(d):
                                vals = jnp.take_along_axis(wc[c], xs, axis=0)
                                outv[b, c, sl, si, pl.ds(bq * 16, 16)] = vals
                    return carry3

                lax.fori_loop(0, _STC, st_body, 0)
                pltpu.make_async_copy(
                    outv.at[b], out_dst(ci), out_sems[b]
                ).start()

                @pl.when(ci + 2 < n_chunks)
                def _():
                    pltpu.make_async_copy(
                        x_src(ci + 2), xv.at[b], in_sems[b]
                    ).start()

            return carry

        lax.fori_loop(0, n_chunks // 2, body, 0)
        for b in range(2):
            pltpu.make_async_copy(
                outv.at[b], out_dst(n_chunks - 2 + b), out_sems[b]
            ).wait()

    return k(wcols, xq)


def kernel(x, weight):
    v, d = weight.shape
    wcols = jnp.zeros((d, 16), jnp.float32).at[:, :v].set(weight.T)
    xq = (
        x.astype(jnp.int32)
        .reshape(_BT, 128, _ST, 8)
        .transpose(2, 0, 3, 1)
    )
    out_q = _embed(xq, wcols)
    return (
        out_q.transpose(2, 4, 1, 3, 0)
        .reshape(16384, 200, d)
    )


# speedup vs baseline: 251.2861x; 1.8031x over previous
import functools

import jax
import jax.numpy as jnp
from jax import lax
from jax.experimental import pallas as pl
from jax.experimental.pallas import tpu as pltpu
from jax.experimental.pallas import tpu_sc as plsc

_NW = 32
_ST = 25
_BT = 128
_BTW = 4
_STC = 5


@jax.jit
def _embed(xq, wcols):
    d = wcols.shape[0]
    mesh = plsc.VectorSubcoreMesh(core_axis_name="c", subcore_axis_name="s")

    n_stc = _ST // _STC
    n_chunks = _BTW * n_stc

    @functools.partial(
        pl.kernel,
        out_type=jax.ShapeDtypeStruct((d, _ST, _BT, 8, 128), jnp.float32),
        mesh=mesh,
        scratch_types=[
            pltpu.VMEM((2, _STC, 8, 128), jnp.int32),
            pltpu.VMEM((2, d, _STC, 8, 128), jnp.float32),
            pltpu.VMEM((d, 16), jnp.float32),
            pltpu.SemaphoreType.DMA,
            pltpu.SemaphoreType.DMA,
            pltpu.SemaphoreType.DMA,
            pltpu.SemaphoreType.DMA,
        ],
        compiler_params=pltpu.CompilerParams(
            use_tc_tiling_on_sc=False, needs_layout_passes=False
        ),
    )
    def k(w_hbm, x_hbm, out_hbm, xv, outv, wv, si0, si1, so0, so1):
        in_sems = (si0, si1)
        out_sems = (so0, so1)
        wid = lax.axis_index("s") * 2 + lax.axis_index("c")
        bt0 = wid * _BTW
        pltpu.sync_copy(w_hbm, wv)
        wc = [wv[c] for c in range(d)]

        def x_src(ci):
            bt = bt0 + ci // n_stc
            st0 = (ci % n_stc) * _STC
            return x_hbm.at[pl.ds(st0, _STC), bt]

        def out_dst(ci):
            bt = bt0 + ci // n_stc
            st0 = (ci % n_stc) * _STC
            return out_hbm.at[:, pl.ds(st0, _STC), bt]

        for b in range(2):
            pltpu.make_async_copy(x_src(b), xv.at[b], in_sems[b]).start()

        def body(j, carry):
            for b in range(2):
                ci = 2 * j + b

                @pl.when(ci >= 2)
                def _():
                    pltpu.make_async_copy(
                        outv.at[b], out_dst(ci - 2), out_sems[b]
                    ).wait()

                pltpu.make_async_copy(x_src(ci), xv.at[b], in_sems[b]).wait()

                @plsc.parallel_loop(0, _STC * 8, unroll=2)
                def st_body(g):
                    sl = g // 8
                    si = g % 8
                    for bq in range(8):
                        xs = xv[b, sl, si, pl.ds(bq * 16, 16)]
                        for c in range(d):
                            vals = jnp.take_along_axis(wc[c], xs, axis=0)
                            outv[b, c, sl, si, pl.ds(bq * 16, 16)] = vals
                pltpu.make_async_copy(
                    outv.at[b], out_dst(ci), out_sems[b]
                ).start()

                @pl.when(ci + 2 < n_chunks)
                def _():
                    pltpu.make_async_copy(
                        x_src(ci + 2), xv.at[b], in_sems[b]
                    ).start()

            return carry

        lax.fori_loop(0, n_chunks // 2, body, 0)
        for b in range(2):
            pltpu.make_async_copy(
                outv.at[b], out_dst(n_chunks - 2 + b), out_sems[b]
            ).wait()

    return k(wcols, xq)


def kernel(x, weight):
    v, d = weight.shape
    wcols = jnp.zeros((d, 16), jnp.float32).at[:, :v].set(weight.T)
    xq = (
        x.astype(jnp.int32)
        .reshape(_BT, 128, _ST, 8)
        .transpose(2, 0, 3, 1)
    )
    out_q = _embed(xq, wcols)
    return (
        out_q.transpose(2, 4, 1, 3, 0)
        .reshape(16384, 200, d)
    )
